# baseline (speedup 1.0000x reference)
import functools

import jax
import jax.numpy as jnp
from jax import lax
from jax.experimental import pallas as pl
from jax.experimental.pallas import tpu as pltpu
from jax.experimental.pallas import tpu_sc as plsc

_N = 32768
_D = 512
_ROWS_BLK = 4096
_NSUB = 16
_CHUNK = _N // _NSUB
_LANES = 16
_NW = 32
_MSC = 4096
_RPW = _MSC // _NW


def _matvec_body(x_ref, w_ref, o_ref):
    o_ref[...] = jnp.dot(x_ref[...], w_ref[...],
                         preferred_element_type=jnp.float32)


def _matvec_tc(feats, w_col):
    nblk = (_N - _MSC) // _ROWS_BLK
    off = _MSC // _ROWS_BLK
    return pl.pallas_call(
        _matvec_body,
        grid=(nblk,),
        in_specs=[
            pl.BlockSpec((_ROWS_BLK, _D), lambda i: (i + off, 0)),
            pl.BlockSpec((_D, 1), lambda i: (0, 0)),
        ],
        out_specs=pl.BlockSpec((_ROWS_BLK, 1), lambda i: (i, 0)),
        out_shape=jax.ShapeDtypeStruct((_N - _MSC, 1), jnp.float32),
        compiler_params=pltpu.CompilerParams(
            dimension_semantics=("arbitrary",)),
    )(feats, w_col)


_mesh = plsc.VectorSubcoreMesh(core_axis_name="c", subcore_axis_name="s")


@functools.partial(
    pl.kernel,
    mesh=_mesh,
    out_type=jax.ShapeDtypeStruct((_MSC,), jnp.float32),
    scratch_types=[
        pltpu.VMEM((_RPW * _D,), jnp.float32),
        pltpu.VMEM((_D,), jnp.float32),
        pltpu.VMEM((_LANES,), jnp.float32),
        pltpu.VMEM((_RPW,), jnp.float32),
    ],
    compiler_params=pltpu.CompilerParams(needs_layout_passes=False),
)
def _sc_matvec(feats1d_hbm, w_hbm, out_hbm, x_v, w_v, acc_r, d_v):
    cid = lax.axis_index("c")
    sid = lax.axis_index("s")
    wid = sid * 2 + cid
    base = wid * _RPW
    pltpu.sync_copy(feats1d_hbm.at[pl.ds(base * _D, _RPW * _D)], x_v)
    pltpu.sync_copy(w_hbm, w_v)
    lanes = lax.iota(jnp.int32, _LANES)
    wj = [w_v[pl.ds(j * _LANES, _LANES)] for j in range(_D // _LANES)]

    def body(r, dv):
        off = r * _D
        acc = x_v[pl.ds(off, _LANES)] * wj[0]
        for j in range(1, _D // _LANES):
            acc = acc + x_v[pl.ds(off + j * _LANES, _LANES)] * wj[j]
        for s in (8, 4, 2, 1):
            acc_r[...] = acc
            acc = acc + plsc.load_gather(acc_r, [lanes ^ s])
        dv = jnp.where(lanes == lax.rem(r, _LANES), acc, dv)

        @pl.when(lax.rem(r, _LANES) == _LANES - 1)
        def _():
            d_v[pl.ds(r - (_LANES - 1), _LANES)] = dv

        return dv

    lax.fori_loop(0, _RPW, body, jnp.zeros((_LANES,), jnp.float32))
    pltpu.sync_copy(d_v, out_hbm.at[pl.ds(base, _RPW)])


@functools.partial(
    pl.kernel,
    mesh=_mesh,
    out_type=jax.ShapeDtypeStruct((2, _D), jnp.float32),
    scratch_types=[
        pltpu.VMEM((_CHUNK,), jnp.float32),
        pltpu.VMEM((_LANES,), jnp.float32),
        pltpu.VMEM((_LANES,), jnp.int32),
        pltpu.VMEM((_LANES,), jnp.float32),
        pltpu.VMEM((_LANES,), jnp.int32),
        pltpu.VMEM_SHARED((_NSUB * _LANES,), jnp.float32),
        pltpu.VMEM_SHARED((_NSUB * _LANES,), jnp.int32),
        pltpu.VMEM_SHARED((_NSUB * _LANES,), jnp.float32),
        pltpu.VMEM_SHARED((_NSUB * _LANES,), jnp.int32),
        pltpu.VMEM((_NSUB * _LANES,), jnp.float32),
        pltpu.VMEM((_NSUB * _LANES,), jnp.int32),
        pltpu.VMEM((_NSUB * _LANES,), jnp.float32),
        pltpu.VMEM((_NSUB * _LANES,), jnp.int32),
        pltpu.VMEM((_LANES,), jnp.int32),
        pltpu.VMEM((_LANES, _D), jnp.float32),
        pltpu.SemaphoreType.DMA,
    ],
    compiler_params=pltpu.CompilerParams(needs_layout_passes=False),
)
def _sc_top1(dsc_hbm, dtc_hbm, feats_hbm, out_hbm, d_v, pvx, pix, pvn, pni,
             shvx, shix, shvn, shni, lvx, lix, lvn, lni, gidx, rows, sem):
    cid = lax.axis_index("c")
    sid = lax.axis_index("s")

    @pl.when(cid == 0)
    def _():
        base = sid * _CHUNK
        nsc = _MSC // _CHUNK

        @pl.when(sid < nsc)
        def _():
            pltpu.sync_copy(dsc_hbm.at[pl.ds(base, _CHUNK)], d_v)

        @pl.when(sid >= nsc)
        def _():
            pltpu.sync_copy(dtc_hbm.at[pl.ds(base - _MSC, _CHUNK)], d_v)

        lanes = lax.iota(jnp.int32, _LANES)
        ninf = jnp.full((_LANES,), -jnp.inf, jnp.float32)
        pinf = jnp.full((_LANES,), jnp.inf, jnp.float32)
        zidx = jnp.zeros((_LANES,), jnp.int32)

        def body(i, carry):
            bvx, bix, bvn, bni = carry
            v = d_v[pl.ds(i * _LANES, _LANES)]
            idx = base + i * _LANES + lanes
            gt = v > bvx
            lt = v < bvn
            return (jnp.where(gt, v, bvx), jnp.where(gt, idx, bix),
                    jnp.where(lt, v, bvn), jnp.where(lt, idx, bni))

        bvx, bix, bvn, bni = lax.fori_loop(
            0, _CHUNK // _LANES, body, (ninf, zidx, pinf, zidx))
        pvx[...] = bvx
        pix[...] = bix
        pvn[...] = bvn
        pni[...] = bni
        off = sid * _LANES
        pltpu.sync_copy(pvx, shvx.at[pl.ds(off, _LANES)])
        pltpu.sync_copy(pix, shix.at[pl.ds(off, _LANES)])
        pltpu.sync_copy(pvn, shvn.at[pl.ds(off, _LANES)])
        pltpu.sync_copy(pni, shni.at[pl.ds(off, _LANES)])
        plsc.subcore_barrier()

        @pl.when(sid == 0)
        def _():
            pltpu.sync_copy(shvx, lvx)
            pltpu.sync_copy(shix, lix)
            pltpu.sync_copy(shvn, lvn)
            pltpu.sync_copy(shni, lni)
            bvx = lvx[pl.ds(0, _LANES)]
            bix = lix[pl.ds(0, _LANES)]
            bvn = lvn[pl.ds(0, _LANES)]
            bni = lni[pl.ds(0, _LANES)]
            for w in range(1, _NSUB):
                v = lvx[pl.ds(w * _LANES, _LANES)]
                ii = lix[pl.ds(w * _LANES, _LANES)]
                gt = v > bvx
                bvx = jnp.where(gt, v, bvx)
                bix = jnp.where(gt, ii, bix)
                v = lvn[pl.ds(w * _LANES, _LANES)]
                ii = lni[pl.ds(w * _LANES, _LANES)]
                lt = v < bvn
                bvn = jnp.where(lt, v, bvn)
                bni = jnp.where(lt, ii, bni)
            for s in (8, 4, 2, 1):
                perm = lanes ^ s
                pvx[...] = bvx
                pix[...] = bix
                pvn[...] = bvn
                pni[...] = bni
                ov = plsc.load_gather(pvx, [perm])
                oi = plsc.load_gather(pix, [perm])
                t = (ov > bvx) | ((ov == bvx) & (oi < bix))
                bvx = jnp.where(t, ov, bvx)
                bix = jnp.where(t, oi, bix)
                ov = plsc.load_gather(pvn, [perm])
                oi = plsc.load_gather(pni, [perm])
                t = (ov < bvn) | ((ov == bvn) & (oi < bni))
                bvn = jnp.where(t, ov, bvn)
                bni = jnp.where(t, oi, bni)
            gv = jnp.where(lanes == 0, bix, jnp.where(lanes == 1, bni, 0))
            gidx[...] = gv
            pltpu.async_copy(feats_hbm.at[gidx], rows, sem).wait()
            pltpu.sync_copy(rows.at[pl.ds(0, 2)], out_hbm)


def kernel(feats, W, b):
    del b
    w_vec = W[0] - W[1]
    d_sc = _sc_matvec(feats.reshape(_N * _D), w_vec)
    d_tc = _matvec_tc(feats, w_vec.reshape(_D, 1)).reshape(_N - _MSC)
    return _sc_top1(d_sc, d_tc, feats)


# speedup vs baseline: 2.2136x; 2.2136x over previous
import functools

import jax
import jax.numpy as jnp
from jax import lax
from jax.experimental import pallas as pl
from jax.experimental.pallas import tpu as pltpu
from jax.experimental.pallas import tpu_sc as plsc

_N = 32768
_D = 512
_ROWS_BLK = 4096
_NBLK = _N // _ROWS_BLK
_LANES = 16
_BIG = 2 ** 30


def _mv_body(x_ref, w_ref, vx_ref, ix_ref, vn_ref, in_ref):
    i = pl.program_id(0)
    d = jnp.dot(x_ref[...], w_ref[...], preferred_element_type=jnp.float32)
    ri = lax.broadcasted_iota(jnp.int32, (_ROWS_BLK, 1), 0)
    big = jnp.int32(_BIG)
    bmax = jnp.max(d)
    bmin = jnp.min(d)
    vx_ref[i] = bmax
    ix_ref[i] = jnp.min(jnp.where(d == bmax, ri, big)) + i * _ROWS_BLK
    vn_ref[i] = bmin
    in_ref[i] = jnp.min(jnp.where(d == bmin, ri, big)) + i * _ROWS_BLK


def _mv_candidates(feats, w_col):
    sd = jax.ShapeDtypeStruct
    return pl.pallas_call(
        _mv_body,
        grid=(_NBLK,),
        in_specs=[
            pl.BlockSpec((_ROWS_BLK, _D), lambda i: (i, 0)),
            pl.BlockSpec((_D, 1), lambda i: (0, 0)),
        ],
        out_specs=[pl.BlockSpec(memory_space=pltpu.SMEM)] * 4,
        out_shape=[sd((_LANES,), jnp.float32), sd((_LANES,), jnp.int32),
                   sd((_LANES,), jnp.float32), sd((_LANES,), jnp.int32)],
        compiler_params=pltpu.CompilerParams(
            dimension_semantics=("arbitrary",)),
    )(feats, w_col)


_mesh = plsc.VectorSubcoreMesh(core_axis_name="c", subcore_axis_name="s")


@functools.partial(
    pl.kernel,
    mesh=_mesh,
    out_type=jax.ShapeDtypeStruct((2, _D), jnp.float32),
    scratch_types=[
        pltpu.VMEM((_LANES,), jnp.float32),
        pltpu.VMEM((_LANES,), jnp.int32),
        pltpu.VMEM((_LANES,), jnp.float32),
        pltpu.VMEM((_LANES,), jnp.int32),
        pltpu.VMEM((_LANES,), jnp.int32),
        pltpu.VMEM((_LANES, _D), jnp.float32),
        pltpu.SemaphoreType.DMA,
    ],
    compiler_params=pltpu.CompilerParams(needs_layout_passes=False),
)
def _sc_select(vx_hbm, ix_hbm, vn_hbm, in_hbm, feats_hbm, out_hbm,
               vx_v, ix_v, vn_v, in_v, gidx, rows, sem):
    cid = lax.axis_index("c")
    sid = lax.axis_index("s")

    @pl.when(jnp.logical_and(cid == 0, sid == 0))
    def _():
        pltpu.sync_copy(vx_hbm, vx_v)
        pltpu.sync_copy(ix_hbm, ix_v)
        pltpu.sync_copy(vn_hbm, vn_v)
        pltpu.sync_copy(in_hbm, in_v)
        lanes = lax.iota(jnp.int32, _LANES)
        valid = lanes < _NBLK
        big = jnp.int32(_BIG)
        bvx = jnp.where(valid, vx_v[...], -jnp.inf)
        bix = jnp.where(valid, ix_v[...], big)
        bvn = jnp.where(valid, vn_v[...], jnp.inf)
        bni = jnp.where(valid, in_v[...], big)
        for s in (8, 4, 2, 1):
            perm = lanes ^ s
            vx_v[...] = bvx
            ix_v[...] = bix
            vn_v[...] = bvn
            in_v[...] = bni
            ov = plsc.load_gather(vx_v, [perm])
            oi = plsc.load_gather(ix_v, [perm])
            t = (ov > bvx) | ((ov == bvx) & (oi < bix))
            bvx = jnp.where(t, ov, bvx)
            bix = jnp.where(t, oi, bix)
            ov = plsc.load_gather(vn_v, [perm])
            oi = plsc.load_gather(in_v, [perm])
            t = (ov < bvn) | ((ov == bvn) & (oi < bni))
            bvn = jnp.where(t, ov, bvn)
            bni = jnp.where(t, oi, bni)
        gidx[...] = jnp.where(lanes == 0, bix, jnp.where(lanes == 1, bni, 0))
        pltpu.async_copy(feats_hbm.at[gidx], rows, sem).wait()
        pltpu.sync_copy(rows.at[pl.ds(0, 2)], out_hbm)


def kernel(feats, W, b):
    del b
    w_col = (W[0] - W[1]).reshape(_D, 1)
    vx, ix, vn, iN = _mv_candidates(feats, w_col)
    return _sc_select(vx, ix, vn, iN, feats)
